# baseline (speedup 1.0000x reference)
import functools

import jax
import jax.numpy as jnp
from jax import lax
from jax.experimental import pallas as pl
from jax.experimental.pallas import tpu as pltpu
from jax.experimental.pallas import tpu_sc as plsc

E = 320000
HID = 128
N_EDGE_TYPE = 16
NSTRIP = HID // 16

_info = plsc.get_sparse_core_info()
_NC, _NS = _info.num_cores, _info.num_subcores
NW = _NC * _NS
R = E // NW
C = 400
NCH = R // C

_mesh = plsc.VectorSubcoreMesh(core_axis_name="c", subcore_axis_name="s")


@functools.partial(
    pl.kernel,
    mesh=_mesh,
    out_type=jax.ShapeDtypeStruct((E, HID), jnp.float32),
    scratch_types=[
        pltpu.VMEM((N_EDGE_TYPE * HID,), jnp.float32),
        pltpu.VMEM((5 * HID,), jnp.float32),
        pltpu.VMEM((C,), jnp.int32),
        pltpu.VMEM((2 * C,), jnp.float32),
        pltpu.VMEM((3 * C,), jnp.float32),
        pltpu.VMEM((2, C, HID), jnp.float32),
        pltpu.SemaphoreType.DMA,
    ],
)
def _sc_embed(et_hbm, rc_hbm, rp_hbm, tbl_hbm, w5_hbm, out_hbm,
              tbl_v, w5_v, et_v, rc_v, rp_v, out_v, sem):
    wid = lax.axis_index("s") * _NC + lax.axis_index("c")
    base = wid * R
    pltpu.sync_copy(tbl_hbm, tbl_v)
    pltpu.sync_copy(w5_hbm, w5_v)
    w = [[w5_v[pl.ds(k * HID + j * 16, 16)] for j in range(NSTRIP)]
         for k in range(5)]

    def chunk(i, carry):
        slot = i % 2
        off = base + i * C
        pltpu.sync_copy(et_hbm.at[pl.ds(off, C)], et_v)
        pltpu.sync_copy(rc_hbm.at[pl.ds(2 * off, 2 * C)], rc_v)
        pltpu.sync_copy(rp_hbm.at[pl.ds(3 * off, 3 * C)], rp_v)

        @pl.when(i >= 2)
        def _():
            pltpu.make_async_copy(out_v.at[slot],
                                  out_hbm.at[pl.ds(base, C)], sem).wait()

        ob = out_v.at[slot]

        def group(g, carry2):
            et16 = et_v[pl.ds(g * 16, 16)]
            rcv = [rc_v[pl.ds(g * 32 + 16 * u, 16)] for u in range(2)]
            rpv = [rp_v[pl.ds(g * 48 + 16 * u, 16)] for u in range(3)]
            for u in range(16):
                t = et16[u]
                a0 = rcv[(2 * u) // 16][(2 * u) % 16]
                a1 = rcv[(2 * u + 1) // 16][(2 * u + 1) % 16]
                a2 = rpv[(3 * u) // 16][(3 * u) % 16]
                a3 = rpv[(3 * u + 1) // 16][(3 * u + 1) % 16]
                a4 = rpv[(3 * u + 2) // 16][(3 * u + 2) % 16]
                tb = t * HID
                e = g * 16 + u
                for j in range(NSTRIP):
                    acc = tbl_v[pl.ds(tb + j * 16, 16)]
                    acc = acc + a0 * w[0][j] + a1 * w[1][j] + a2 * w[2][j]
                    acc = acc + a3 * w[3][j] + a4 * w[4][j]
                    ob[e, pl.ds(j * 16, 16)] = acc
            return carry2

        lax.fori_loop(0, C // 16, group, 0)
        pltpu.async_copy(ob, out_hbm.at[pl.ds(off, C)], sem)
        return carry

    lax.fori_loop(0, NCH, chunk, 0)
    pltpu.make_async_copy(out_v.at[0], out_hbm.at[pl.ds(base, C)], sem).wait()
    pltpu.make_async_copy(out_v.at[1], out_hbm.at[pl.ds(base, C)], sem).wait()


@jax.jit
def kernel(edge_type, att_rc, att_rp, type_table, W_rc, b_rc, W_rp, b_rp):
    tbl2 = (type_table + b_rc + b_rp).reshape(-1)
    w5 = jnp.concatenate([W_rc, W_rp], axis=0).reshape(-1)
    et = edge_type.astype(jnp.int32)
    return _sc_embed(et, att_rc.reshape(-1), att_rp.reshape(-1), tbl2, w5)
